# baseline (speedup 1.0000x reference)
import jax
import jax.numpy as jnp
from jax.experimental import pallas as pl
from jax.experimental.pallas import tpu as pltpu

_M = 4096
_K = 128
_N = 100000
_M_BLK = 256
_N_BLK = 128
_G_M = _M // _M_BLK
_G_N = (_N + _N_BLK - 1) // _N_BLK
_N_PAD = _G_N * _N_BLK
_BIG = 2**31 - 1


def _centroid_kernel(latent_ref, coords_ref, out_ref, bestv_ref, besti_ref):
    n = pl.program_id(0)
    m = pl.program_id(1)
    lat = latent_ref[...]
    cb = coords_ref[...]
    mm = jax.lax.dot_general(
        lat.astype(jnp.bfloat16), cb.astype(jnp.bfloat16),
        (((1,), (1,)), ((), ())),
        preferred_element_type=jnp.float32)
    a2 = jnp.sum(lat * lat, axis=1)
    b2 = jnp.sum(cb * cb, axis=1)
    d2 = jnp.maximum((a2[:, None] + b2[None, :]) - 2.0 * mm, 0.0)
    score = d2 * jax.lax.rsqrt(d2)
    score = jnp.where(d2 == 0.0, 0.0, score)
    cand = jax.lax.broadcasted_iota(jnp.int32, (_M_BLK, _N_BLK), 1) + n * _N_BLK

    sl = pl.ds(m * _M_BLK, _M_BLK)

    @pl.when(n == 0)
    def _init():
        bestv_ref[sl, :] = score
        besti_ref[sl, :] = cand

    @pl.when(n > 0)
    def _merge():
        pv = bestv_ref[sl, :]
        pi = besti_ref[sl, :]
        better = score < pv
        bestv_ref[sl, :] = jnp.where(better, score, pv)
        besti_ref[sl, :] = jnp.where(better, cand, pi)

    @pl.when(n == _G_N - 1)
    def _emit():
        vals_t = jnp.transpose(bestv_ref[sl, :])
        idxs_t = jnp.transpose(besti_ref[sl, :])
        minv = jnp.min(vals_t, axis=0)
        eq = vals_t == minv[None, :]
        out_ref[...] = jnp.min(jnp.where(eq, idxs_t, _BIG), axis=0)


@jax.jit
def kernel(latent, coords):
    coords_p = jnp.pad(coords, ((0, _N_PAD - _N), (0, 0)),
                       constant_values=1e6)
    out = pl.pallas_call(
        _centroid_kernel,
        grid=(_G_N, _G_M),
        in_specs=[
            pl.BlockSpec((_M_BLK, _K), lambda n, m: (m, 0)),
            pl.BlockSpec((_N_BLK, _K), lambda n, m: (n, 0)),
        ],
        out_specs=pl.BlockSpec((_M_BLK,), lambda n, m: (m,)),
        out_shape=jax.ShapeDtypeStruct((_M,), jnp.int32),
        scratch_shapes=[
            pltpu.VMEM((_M, 128), jnp.float32),
            pltpu.VMEM((_M, 128), jnp.int32),
        ],
        compiler_params=pltpu.CompilerParams(
            dimension_semantics=("arbitrary", "arbitrary"),
        ),
    )(latent, coords_p)
    return out
